# baseline (speedup 1.0000x reference)
import functools

import jax
import jax.numpy as jnp
from jax import lax
from jax.experimental import pallas as pl
from jax.experimental.pallas import tpu as pltpu
from jax.experimental.pallas import tpu_sc as plsc

D = 128
NC, NS = 2, 16
NW = NC * NS
CHUNK = 40
NBUF = 3


def _prep_body(iw_ref, t0_ref, t1_ref, f_ref, e_ref, a_ref, idx2_ref):
    t0 = t0_ref[...]
    t1 = t1_ref[...]
    h0 = jnp.tanh(t0)
    h1 = jnp.tanh(t1)
    s0 = 1.0 / (1.0 + jnp.exp(h1 - h0))
    s1 = 1.0 - s0
    f_ref[...] = t0 * s0 + t1 * s1
    v = t0.shape[0]
    e_ref[0:v, :] = t0
    e_ref[v:2 * v, :] = t1
    a_ref[0:v, :] = s0
    a_ref[v:2 * v, :] = s1
    l = iw_ref.shape[1]
    rows = lax.broadcasted_iota(jnp.int32, (l, 2 * l), 0)
    cols = lax.broadcasted_iota(jnp.int32, (l, 2 * l), 1)
    perm = (cols // 2 == rows).astype(jnp.float32)
    dup = lax.dot(iw_ref[...].astype(jnp.float32), perm,
                  precision=lax.Precision.HIGHEST)
    odd = (lax.broadcasted_iota(jnp.int32, dup.shape, 1) % 2) * v
    idx2_ref[...] = dup.astype(jnp.int32) + odd


def _prep(iw, t0, t1):
    v = t0.shape[0]
    b, l = iw.shape
    return pl.pallas_call(
        _prep_body,
        out_shape=(
            jax.ShapeDtypeStruct((v, D), jnp.float32),
            jax.ShapeDtypeStruct((2 * v, D), jnp.float32),
            jax.ShapeDtypeStruct((2 * v, D), jnp.float32),
            jax.ShapeDtypeStruct((b, 2 * l), jnp.int32),
        ),
    )(iw, t0, t1)


@functools.lru_cache(maxsize=None)
def _make_gather(b_total):
    b_per_w = b_total // NW
    n = b_per_w // CHUNK
    assert b_per_w * NW == b_total and n * CHUNK == b_per_w

    @functools.partial(
        pl.kernel,
        out_type=(
            jax.ShapeDtypeStruct((b_total, D), jnp.float32),
            jax.ShapeDtypeStruct((2 * b_total, D), jnp.float32),
            jax.ShapeDtypeStruct((2 * b_total, D), jnp.float32),
        ),
        mesh=plsc.VectorSubcoreMesh(core_axis_name="c", subcore_axis_name="s"),
        scratch_types=[
            pltpu.VMEM((n, 2 * CHUNK), jnp.int32),
            pltpu.VMEM((n, CHUNK), jnp.int32),
            pltpu.VMEM((NBUF, CHUNK, D), jnp.float32),
            pltpu.VMEM((NBUF, 2 * CHUNK, D), jnp.float32),
            pltpu.VMEM((NBUF, 2 * CHUNK, D), jnp.float32),
            pltpu.SemaphoreType.DMA,
            pltpu.SemaphoreType.DMA,
        ],
    )
    def gather(idxb_hbm, idxl_hbm, f_hbm, e_hbm, a_hbm, of_hbm, oe_hbm, oa_hbm,
               idxb_v, idxl_v, bf, be, ba, gsem, ssem):
        wid = lax.axis_index("s") * NC + lax.axis_index("c")

        def fire_gathers(i, p):
            pltpu.async_copy(f_hbm.at[idxl_v.at[i]], bf.at[p], gsem)
            pltpu.async_copy(e_hbm.at[idxb_v.at[i]], be.at[p], gsem)
            pltpu.async_copy(a_hbm.at[idxb_v.at[i]], ba.at[p], gsem)

        def drain_gathers(p):
            pltpu.make_async_copy(f_hbm.at[pl.ds(0, CHUNK)], bf.at[p], gsem).wait()
            pltpu.make_async_copy(e_hbm.at[pl.ds(0, 2 * CHUNK)], be.at[p], gsem).wait()
            pltpu.make_async_copy(a_hbm.at[pl.ds(0, 2 * CHUNK)], ba.at[p], gsem).wait()

        def fire_stores(i, p):
            base = wid * b_per_w + i * CHUNK
            pltpu.async_copy(bf.at[p], of_hbm.at[pl.ds(base, CHUNK)], ssem)
            pltpu.async_copy(be.at[p], oe_hbm.at[pl.ds(2 * base, 2 * CHUNK)], ssem)
            pltpu.async_copy(ba.at[p], oa_hbm.at[pl.ds(2 * base, 2 * CHUNK)], ssem)

        def drain_stores(p):
            pltpu.make_async_copy(bf.at[p], of_hbm.at[pl.ds(0, CHUNK)], ssem).wait()
            pltpu.make_async_copy(be.at[p], oe_hbm.at[pl.ds(0, 2 * CHUNK)], ssem).wait()
            pltpu.make_async_copy(ba.at[p], oa_hbm.at[pl.ds(0, 2 * CHUNK)], ssem).wait()

        pltpu.sync_copy(idxb_hbm.at[wid], idxb_v)
        pltpu.sync_copy(idxl_hbm.at[wid], idxl_v)
        fire_gathers(0, 0)
        fire_gathers(1, 1)

        def body(i, carry):
            @pl.when(i >= 1)
            def _():
                drain_stores((i - 1) % NBUF)

            @pl.when(i + 2 < n)
            def _():
                fire_gathers(i + 2, (i + 2) % NBUF)

            drain_gathers(i % NBUF)
            fire_stores(i, i % NBUF)
            return carry

        lax.fori_loop(0, n, body, 0)
        drain_stores((n - 1) % NBUF)

    return gather


def kernel(input_words, T0, T1):
    b, l = input_words.shape
    iw = input_words.astype(jnp.int32)
    n = (b * l) // (NW * CHUNK)
    f_tab, e_tab, a_tab, idx2 = _prep(iw, T0, T1)
    idxb = idx2.reshape(NW, n, 2 * CHUNK)
    idxl = iw.T.reshape(NW, n, CHUNK)
    of, oe, oa = _make_gather(b * l)(idxb, idxl, f_tab, e_tab, a_tab)
    final = of.reshape(l, b, D).transpose(1, 0, 2)
    emb = oe.reshape(b, l, 2, D).transpose(0, 1, 3, 2)
    attn = oa.reshape(b, l, 2, D).transpose(0, 1, 3, 2)
    return (final, emb, attn)


# speedup vs baseline: 1.0169x; 1.0169x over previous
import functools

import jax
import jax.numpy as jnp
from jax import lax
from jax.experimental import pallas as pl
from jax.experimental.pallas import tpu as pltpu
from jax.experimental.pallas import tpu_sc as plsc

D = 128
NC, NS = 2, 16
NW = NC * NS
CHUNK = 64


def _prep_body(iw_ref, t0_ref, t1_ref, f_ref, e_ref, a_ref, idx2_ref):
    t0 = t0_ref[...]
    t1 = t1_ref[...]
    h0 = jnp.tanh(t0)
    h1 = jnp.tanh(t1)
    s0 = 1.0 / (1.0 + jnp.exp(h1 - h0))
    s1 = 1.0 - s0
    f_ref[...] = t0 * s0 + t1 * s1
    v = t0.shape[0]
    e_ref[0:v, :] = t0
    e_ref[v:2 * v, :] = t1
    a_ref[0:v, :] = s0
    a_ref[v:2 * v, :] = s1
    l = iw_ref.shape[1]
    rows = lax.broadcasted_iota(jnp.int32, (l, 2 * l), 0)
    cols = lax.broadcasted_iota(jnp.int32, (l, 2 * l), 1)
    perm = (cols // 2 == rows).astype(jnp.float32)
    dup = lax.dot(iw_ref[...].astype(jnp.float32), perm,
                  precision=lax.Precision.HIGHEST)
    odd = (lax.broadcasted_iota(jnp.int32, dup.shape, 1) % 2) * v
    idx2_ref[...] = dup.astype(jnp.int32) + odd


def _prep(iw, t0, t1):
    v = t0.shape[0]
    b, l = iw.shape
    return pl.pallas_call(
        _prep_body,
        out_shape=(
            jax.ShapeDtypeStruct((v, D), jnp.float32),
            jax.ShapeDtypeStruct((2 * v, D), jnp.float32),
            jax.ShapeDtypeStruct((2 * v, D), jnp.float32),
            jax.ShapeDtypeStruct((b, 2 * l), jnp.int32),
        ),
    )(iw, t0, t1)


@functools.lru_cache(maxsize=None)
def _make_gather(b_total):
    b_per_w = b_total // NW
    n = b_per_w // CHUNK
    assert b_per_w * NW == b_total and n * CHUNK == b_per_w

    @functools.partial(
        pl.kernel,
        out_type=(
            jax.ShapeDtypeStruct((b_total, D), jnp.float32),
            jax.ShapeDtypeStruct((2 * b_total, D), jnp.float32),
            jax.ShapeDtypeStruct((2 * b_total, D), jnp.float32),
        ),
        mesh=plsc.VectorSubcoreMesh(core_axis_name="c", subcore_axis_name="s"),
        scratch_types=[
            pltpu.VMEM((n, 2 * CHUNK), jnp.int32),
            pltpu.VMEM((n, CHUNK), jnp.int32),
            pltpu.VMEM((2, CHUNK, D), jnp.float32),
            pltpu.VMEM((2, 2 * CHUNK, D), jnp.float32),
            pltpu.VMEM((2, 2 * CHUNK, D), jnp.float32),
            pltpu.SemaphoreType.DMA,
            pltpu.SemaphoreType.DMA,
        ],
    )
    def gather(idxb_hbm, idxl_hbm, f_hbm, e_hbm, a_hbm, of_hbm, oe_hbm, oa_hbm,
               idxb_v, idxl_v, bf, be, ba, gsem, ssem):
        wid = lax.axis_index("s") * NC + lax.axis_index("c")

        def fire_gathers(i, p):
            pltpu.async_copy(f_hbm.at[idxl_v.at[i]], bf.at[p], gsem)
            pltpu.async_copy(e_hbm.at[idxb_v.at[i]], be.at[p], gsem)
            pltpu.async_copy(a_hbm.at[idxb_v.at[i]], ba.at[p], gsem)

        def drain_gathers(p):
            pltpu.make_async_copy(f_hbm.at[pl.ds(0, CHUNK)], bf.at[p], gsem).wait()
            pltpu.make_async_copy(e_hbm.at[pl.ds(0, 2 * CHUNK)], be.at[p], gsem).wait()
            pltpu.make_async_copy(a_hbm.at[pl.ds(0, 2 * CHUNK)], ba.at[p], gsem).wait()

        def fire_stores(i, p):
            base = wid * b_per_w + i * CHUNK
            pltpu.async_copy(bf.at[p], of_hbm.at[pl.ds(base, CHUNK)], ssem)
            pltpu.async_copy(be.at[p], oe_hbm.at[pl.ds(2 * base, 2 * CHUNK)], ssem)
            pltpu.async_copy(ba.at[p], oa_hbm.at[pl.ds(2 * base, 2 * CHUNK)], ssem)

        def drain_stores(p):
            pltpu.make_async_copy(bf.at[p], of_hbm.at[pl.ds(0, CHUNK)], ssem).wait()
            pltpu.make_async_copy(be.at[p], oe_hbm.at[pl.ds(0, 2 * CHUNK)], ssem).wait()
            pltpu.make_async_copy(ba.at[p], oa_hbm.at[pl.ds(0, 2 * CHUNK)], ssem).wait()

        pltpu.sync_copy(idxb_hbm.at[wid], idxb_v)
        pltpu.sync_copy(idxl_hbm.at[wid], idxl_v)
        fire_gathers(0, 0)

        def body(i, carry):
            p = i % 2
            q = (i + 1) % 2

            @pl.when(i > 0)
            def _():
                drain_stores(q)

            @pl.when(i < n - 1)
            def _():
                fire_gathers(i + 1, q)

            drain_gathers(p)
            fire_stores(i, p)
            return carry

        lax.fori_loop(0, n, body, 0)
        drain_stores((n - 1) % 2)

    return gather


def kernel(input_words, T0, T1):
    b, l = input_words.shape
    iw = input_words.astype(jnp.int32)
    n = (b * l) // (NW * CHUNK)
    f_tab, e_tab, a_tab, idx2 = _prep(iw, T0, T1)
    idxb = idx2.reshape(NW, n, 2 * CHUNK)
    idxl = iw.T.reshape(NW, n, CHUNK)
    of, oe, oa = _make_gather(b * l)(idxb, idxl, f_tab, e_tab, a_tab)
    final = of.reshape(l, b, D).transpose(1, 0, 2)
    emb = oe.reshape(b, l, 2, D).transpose(0, 1, 3, 2)
    attn = oa.reshape(b, l, 2, D).transpose(0, 1, 3, 2)
    return (final, emb, attn)


# speedup vs baseline: 1.0185x; 1.0016x over previous
import functools

import jax
import jax.numpy as jnp
from jax import lax
from jax.experimental import pallas as pl
from jax.experimental.pallas import tpu as pltpu
from jax.experimental.pallas import tpu_sc as plsc

D = 128
NC, NS = 2, 16
NW = NC * NS
CHUNK = 64


def _prep_body(iw_ref, t0_ref, t1_ref, f_ref, e_ref, a_ref, idx2_ref):
    t0 = t0_ref[...]
    t1 = t1_ref[...]
    h0 = jnp.tanh(t0)
    h1 = jnp.tanh(t1)
    s0 = 1.0 / (1.0 + jnp.exp(h1 - h0))
    s1 = 1.0 - s0
    f_ref[...] = t0 * s0 + t1 * s1
    v = t0.shape[0]
    e_ref[0:v, :] = t0
    e_ref[v:2 * v, :] = t1
    a_ref[0:v, :] = s0
    a_ref[v:2 * v, :] = s1
    l = iw_ref.shape[1]
    rows = lax.broadcasted_iota(jnp.int32, (l, 2 * l), 0)
    cols = lax.broadcasted_iota(jnp.int32, (l, 2 * l), 1)
    perm = (cols // 2 == rows).astype(jnp.float32)
    dup = lax.dot(iw_ref[...].astype(jnp.float32), perm,
                  precision=lax.Precision.HIGHEST)
    odd = (lax.broadcasted_iota(jnp.int32, dup.shape, 1) % 2) * v
    idx2_ref[...] = dup.astype(jnp.int32) + odd


def _prep(iw, t0, t1):
    v = t0.shape[0]
    b, l = iw.shape
    return pl.pallas_call(
        _prep_body,
        out_shape=(
            jax.ShapeDtypeStruct((v, D), jnp.float32),
            jax.ShapeDtypeStruct((2 * v, D), jnp.float32),
            jax.ShapeDtypeStruct((2 * v, D), jnp.float32),
            jax.ShapeDtypeStruct((b, 2 * l), jnp.int32),
        ),
    )(iw, t0, t1)


@functools.lru_cache(maxsize=None)
def _make_gather(b_total):
    b_per_w = b_total // NW
    n = b_per_w // CHUNK
    assert b_per_w * NW == b_total and n * CHUNK == b_per_w

    @functools.partial(
        pl.kernel,
        out_type=(
            jax.ShapeDtypeStruct((b_total, D), jnp.float32),
            jax.ShapeDtypeStruct((2 * b_total, D), jnp.float32),
            jax.ShapeDtypeStruct((2 * b_total, D), jnp.float32),
        ),
        mesh=plsc.VectorSubcoreMesh(core_axis_name="c", subcore_axis_name="s"),
        scratch_types=[
            pltpu.VMEM((n, 2 * CHUNK), jnp.int32),
            pltpu.VMEM((n, CHUNK), jnp.int32),
            pltpu.VMEM((2, CHUNK, D), jnp.float32),
            pltpu.VMEM((2, 2 * CHUNK, D), jnp.float32),
            pltpu.VMEM((2, 2 * CHUNK, D), jnp.float32),
            pltpu.SemaphoreType.DMA,
            pltpu.SemaphoreType.DMA,
            pltpu.SemaphoreType.DMA,
            pltpu.SemaphoreType.DMA,
            pltpu.SemaphoreType.DMA,
            pltpu.SemaphoreType.DMA,
        ],
    )
    def gather(idxb_hbm, idxl_hbm, f_hbm, e_hbm, a_hbm, of_hbm, oe_hbm, oa_hbm,
               idxb_v, idxl_v, bf, be, ba, gf, ge, ga, sf, se, sa):
        wid = lax.axis_index("s") * NC + lax.axis_index("c")

        def fire_gathers(i, p):
            pltpu.async_copy(f_hbm.at[idxl_v.at[i]], bf.at[p], gf)
            pltpu.async_copy(e_hbm.at[idxb_v.at[i]], be.at[p], ge)
            pltpu.async_copy(a_hbm.at[idxb_v.at[i]], ba.at[p], ga)

        def store_base(i):
            return wid * b_per_w + i * CHUNK

        def stream_f(i, p):
            pltpu.make_async_copy(f_hbm.at[pl.ds(0, CHUNK)], bf.at[p], gf).wait()
            pltpu.async_copy(bf.at[p], of_hbm.at[pl.ds(store_base(i), CHUNK)], sf)

        def stream_e(i, p):
            pltpu.make_async_copy(e_hbm.at[pl.ds(0, 2 * CHUNK)], be.at[p], ge).wait()
            pltpu.async_copy(be.at[p],
                             oe_hbm.at[pl.ds(2 * store_base(i), 2 * CHUNK)], se)

        def stream_a(i, p):
            pltpu.make_async_copy(a_hbm.at[pl.ds(0, 2 * CHUNK)], ba.at[p], ga).wait()
            pltpu.async_copy(ba.at[p],
                             oa_hbm.at[pl.ds(2 * store_base(i), 2 * CHUNK)], sa)

        def drain_stores(p):
            pltpu.make_async_copy(bf.at[p], of_hbm.at[pl.ds(0, CHUNK)], sf).wait()
            pltpu.make_async_copy(be.at[p], oe_hbm.at[pl.ds(0, 2 * CHUNK)], se).wait()
            pltpu.make_async_copy(ba.at[p], oa_hbm.at[pl.ds(0, 2 * CHUNK)], sa).wait()

        pltpu.sync_copy(idxb_hbm.at[wid], idxb_v)
        pltpu.sync_copy(idxl_hbm.at[wid], idxl_v)
        fire_gathers(0, 0)

        def body(i, carry):
            p = i % 2
            q = (i + 1) % 2

            @pl.when(i > 0)
            def _():
                drain_stores(q)

            @pl.when(i < n - 1)
            def _():
                fire_gathers(i + 1, q)

            stream_f(i, p)
            stream_e(i, p)
            stream_a(i, p)
            return carry

        lax.fori_loop(0, n, body, 0)
        drain_stores((n - 1) % 2)

    return gather


def kernel(input_words, T0, T1):
    b, l = input_words.shape
    iw = input_words.astype(jnp.int32)
    n = (b * l) // (NW * CHUNK)
    f_tab, e_tab, a_tab, idx2 = _prep(iw, T0, T1)
    idxb = idx2.reshape(NW, n, 2 * CHUNK)
    idxl = iw.T.reshape(NW, n, CHUNK)
    of, oe, oa = _make_gather(b * l)(idxb, idxl, f_tab, e_tab, a_tab)
    final = of.reshape(l, b, D).transpose(1, 0, 2)
    emb = oe.reshape(b, l, 2, D).transpose(0, 1, 3, 2)
    attn = oa.reshape(b, l, 2, D).transpose(0, 1, 3, 2)
    return (final, emb, attn)
